# baseline (speedup 1.0000x reference)
import functools

import jax
import jax.numpy as jnp
from jax import lax
from jax.experimental import pallas as pl
from jax.experimental.pallas import tpu as pltpu
from jax.experimental.pallas import tpu_sc as plsc

N = 10000
D = 256
DH = 128
E = 160000
E_PAD = 163840
ACC_ROWS = 10112
ROWS_PER_TILE = 632
NC, NS = 2, 16
CHUNK = 128
N_HALF = 2
HALF_CHUNKS = E_PAD // NS // N_HALF // CHUNK
DEG_CHUNKS = E_PAD // (NC * NS) // CHUNK
NBLK = 400
NSTEPS = N // NBLK

_mesh = plsc.VectorSubcoreMesh(core_axis_name="c", subcore_axis_name="s")

_acc_sd = jax.ShapeDtypeStruct((ACC_ROWS, DH), jnp.float32)


@functools.partial(
    pl.kernel, mesh=_mesh,
    out_type=(_acc_sd, _acc_sd),
    scratch_types=[
        pltpu.VMEM((DEG_CHUNKS, CHUNK), jnp.int32),
        pltpu.VMEM((CHUNK, DH), jnp.float32),
        pltpu.VMEM_SHARED((ACC_ROWS, DH), jnp.float32),
        pltpu.SemaphoreType.DMA,
    ],
)
def _deg_kernel(dst_hbm, ones_hbm, zeros_hbm, deg0_hbm, deg1_hbm,
                idx_v, ones_v, acc, dsem):
    c = lax.axis_index("c")
    s = lax.axis_index("s")
    wid = c * NS + s
    sl = pl.ds(s * ROWS_PER_TILE, ROWS_PER_TILE)
    pltpu.sync_copy(dst_hbm.at[wid], idx_v)
    pltpu.sync_copy(ones_hbm, ones_v)
    pltpu.sync_copy(zeros_hbm, acc.at[sl])
    plsc.subcore_barrier()

    @pl.loop(0, DEG_CHUNKS)
    def _(j):
        pltpu.async_copy(ones_v, acc.at[idx_v.at[j]], dsem, add=True)

    @pl.loop(0, DEG_CHUNKS)
    def _(j):
        pltpu.make_async_copy(ones_v, acc.at[idx_v.at[j]], dsem).wait()

    plsc.subcore_barrier()

    def out_copy(dref):
        pltpu.sync_copy(acc.at[sl], dref.at[sl])

    pl.when(c == 0)(lambda: out_copy(deg0_hbm))
    pl.when(c == 1)(lambda: out_copy(deg1_hbm))


@functools.partial(
    pl.kernel, mesh=_mesh,
    out_type=(_acc_sd, _acc_sd),
    scratch_types=[
        pltpu.VMEM((HALF_CHUNKS, CHUNK), jnp.int32),
        pltpu.VMEM((HALF_CHUNKS, CHUNK), jnp.int32),
        pltpu.VMEM((CHUNK, DH), jnp.float32),
        pltpu.VMEM((CHUNK, DH), jnp.float32),
        pltpu.VMEM_SHARED((ACC_ROWS, DH), jnp.float32),
        pltpu.SemaphoreType.DMA,
        pltpu.SemaphoreType.DMA,
        pltpu.SemaphoreType.DMA,
        pltpu.SemaphoreType.DMA,
    ],
)
def _agg_kernel(g0_hbm, g1_hbm, src_hbm, dst_hbm, zeros_hbm,
                out0_hbm, out1_hbm, idx_s, idx_d, buf0, buf1, acc,
                sem0, sem1, ssem0, ssem1):
    c = lax.axis_index("c")
    s = lax.axis_index("s")
    sl = pl.ds(s * ROWS_PER_TILE, ROWS_PER_TILE)
    pltpu.sync_copy(zeros_hbm, acc.at[sl])
    plsc.subcore_barrier()

    def run(g_hbm):
        @pl.loop(0, N_HALF)
        def _(h):
            pltpu.sync_copy(src_hbm.at[s * N_HALF + h], idx_s)
            pltpu.sync_copy(dst_hbm.at[s * N_HALF + h], idx_d)
            pltpu.async_copy(g_hbm.at[idx_s.at[0]], buf0, sem0)
            pltpu.async_copy(g_hbm.at[idx_s.at[1]], buf1, sem1)

            @pl.loop(0, HALF_CHUNKS, step=2)
            def _(j):
                pltpu.make_async_copy(g_hbm.at[idx_s.at[j]], buf0, sem0).wait()
                hs0 = pltpu.async_copy(buf0, acc.at[idx_d.at[j]], ssem0,
                                       add=True)
                pltpu.make_async_copy(g_hbm.at[idx_s.at[j + 1]], buf1,
                                      sem1).wait()
                hs1 = pltpu.async_copy(buf1, acc.at[idx_d.at[j + 1]], ssem1,
                                       add=True)
                hs0.wait()

                @pl.when(j + 2 < HALF_CHUNKS)
                def _():
                    pltpu.async_copy(g_hbm.at[idx_s.at[j + 2]], buf0, sem0)

                hs1.wait()

                @pl.when(j + 3 < HALF_CHUNKS)
                def _():
                    pltpu.async_copy(g_hbm.at[idx_s.at[j + 3]], buf1, sem1)

    pl.when(c == 0)(lambda: run(g0_hbm))
    pl.when(c == 1)(lambda: run(g1_hbm))
    plsc.subcore_barrier()

    def out_copy(oref):
        pltpu.sync_copy(acc.at[sl], oref.at[sl])

    pl.when(c == 0)(lambda: out_copy(out0_hbm))
    pl.when(c == 1)(lambda: out_copy(out1_hbm))


def _dis_body(d0_ref, d1_ref, o_ref):
    deg = d0_ref[:, 0:1] + d1_ref[:, 0:1] + 1.0
    o_ref[...] = lax.rsqrt(deg)


def _mm1_body(x_ref, w_ref, dis_ref, g0_ref, g1_ref):
    h = jnp.dot(x_ref[...], w_ref[...], preferred_element_type=jnp.float32)
    g = h * dis_ref[...]
    g0_ref[...] = g[:, :DH]
    g1_ref[...] = g[:, DH:]


def _mid_body(a0_ref, a1_ref, g0_ref, g1_ref, dis_ref, b1_ref, w2_ref,
              o0_ref, o1_ref):
    agg = jnp.concatenate([a0_ref[...], a1_ref[...]], axis=1)
    g = jnp.concatenate([g0_ref[...], g1_ref[...]], axis=1)
    z = jnp.maximum(dis_ref[...] * (agg + g) + b1_ref[...], 0.0)
    h2 = jnp.dot(z, w2_ref[...], preferred_element_type=jnp.float32)
    g2 = h2 * dis_ref[...]
    o0_ref[...] = g2[:, :DH]
    o1_ref[...] = g2[:, DH:]


def _fin_body(a0_ref, a1_ref, g0_ref, g1_ref, dis_ref, b2_ref, o_ref):
    i = pl.program_id(0)
    agg = jnp.concatenate([a0_ref[...], a1_ref[...]], axis=1)
    g = jnp.concatenate([g0_ref[...], g1_ref[...]], axis=1)
    y = dis_ref[...] * (agg + g)
    part = jnp.sum(y, axis=0, keepdims=True)

    @pl.when(i == 0)
    def _():
        o_ref[...] = part

    @pl.when(i > 0)
    def _():
        o_ref[...] += part

    @pl.when(i == NSTEPS - 1)
    def _():
        o_ref[...] = o_ref[...] * (1.0 / N) + b2_ref[...]


_dis_call = pl.pallas_call(
    _dis_body,
    grid=(8,),
    in_specs=[
        pl.BlockSpec((ACC_ROWS // 8, DH), lambda i: (i, 0)),
        pl.BlockSpec((ACC_ROWS // 8, DH), lambda i: (i, 0)),
    ],
    out_specs=pl.BlockSpec((ACC_ROWS // 8, 1), lambda i: (i, 0)),
    out_shape=jax.ShapeDtypeStruct((ACC_ROWS, 1), jnp.float32),
)

_mm1_call = pl.pallas_call(
    _mm1_body,
    grid=(NSTEPS,),
    in_specs=[
        pl.BlockSpec((NBLK, D), lambda i: (i, 0)),
        pl.BlockSpec((D, D), lambda i: (0, 0)),
        pl.BlockSpec((NBLK, 1), lambda i: (i, 0)),
    ],
    out_specs=(
        pl.BlockSpec((NBLK, DH), lambda i: (i, 0)),
        pl.BlockSpec((NBLK, DH), lambda i: (i, 0)),
    ),
    out_shape=(
        jax.ShapeDtypeStruct((N, DH), jnp.float32),
        jax.ShapeDtypeStruct((N, DH), jnp.float32),
    ),
)

_mid_call = pl.pallas_call(
    _mid_body,
    grid=(NSTEPS,),
    in_specs=[
        pl.BlockSpec((NBLK, DH), lambda i: (i, 0)),
        pl.BlockSpec((NBLK, DH), lambda i: (i, 0)),
        pl.BlockSpec((NBLK, DH), lambda i: (i, 0)),
        pl.BlockSpec((NBLK, DH), lambda i: (i, 0)),
        pl.BlockSpec((NBLK, 1), lambda i: (i, 0)),
        pl.BlockSpec((1, D), lambda i: (0, 0)),
        pl.BlockSpec((D, D), lambda i: (0, 0)),
    ],
    out_specs=(
        pl.BlockSpec((NBLK, DH), lambda i: (i, 0)),
        pl.BlockSpec((NBLK, DH), lambda i: (i, 0)),
    ),
    out_shape=(
        jax.ShapeDtypeStruct((N, DH), jnp.float32),
        jax.ShapeDtypeStruct((N, DH), jnp.float32),
    ),
)

_fin_call = pl.pallas_call(
    _fin_body,
    grid=(NSTEPS,),
    in_specs=[
        pl.BlockSpec((NBLK, DH), lambda i: (i, 0)),
        pl.BlockSpec((NBLK, DH), lambda i: (i, 0)),
        pl.BlockSpec((NBLK, DH), lambda i: (i, 0)),
        pl.BlockSpec((NBLK, DH), lambda i: (i, 0)),
        pl.BlockSpec((NBLK, 1), lambda i: (i, 0)),
        pl.BlockSpec((1, D), lambda i: (0, 0)),
    ],
    out_specs=pl.BlockSpec((1, D), lambda i: (0, 0)),
    out_shape=jax.ShapeDtypeStruct((1, D), jnp.float32),
)


@jax.jit
def kernel(x, edge_index, W1, b1, W2, b2):
    src = edge_index[0].astype(jnp.int32)
    dst = edge_index[1].astype(jnp.int32)
    pad = E_PAD - E
    src_p = jnp.concatenate([src, jnp.zeros((pad,), jnp.int32)])
    dst_p = jnp.concatenate([dst, jnp.full((pad,), N, jnp.int32)])
    src_agg = src_p.reshape(NS * N_HALF, HALF_CHUNKS, CHUNK)
    dst_agg = dst_p.reshape(NS * N_HALF, HALF_CHUNKS, CHUNK)
    dst_deg = dst_p.reshape(NC * NS, DEG_CHUNKS, CHUNK)
    zeros = jnp.zeros((ROWS_PER_TILE, DH), jnp.float32)
    ones = jnp.ones((CHUNK, DH), jnp.float32)

    deg0, deg1 = _deg_kernel(dst_deg, ones, zeros)
    dis = _dis_call(deg0, deg1)

    g0, g1 = _mm1_call(x, W1, dis)
    a0, a1 = _agg_kernel(g0, g1, src_agg, dst_agg, zeros)
    h0, h1 = _mid_call(a0, a1, g0, g1, dis, b1.reshape(1, D), W2)
    c0, c1 = _agg_kernel(h0, h1, src_agg, dst_agg, zeros)
    out = _fin_call(c0, c1, h0, h1, dis, b2.reshape(1, D))
    return out


# speedup vs baseline: 1.0683x; 1.0683x over previous
import functools

import jax
import jax.numpy as jnp
from jax import lax
from jax.experimental import pallas as pl
from jax.experimental.pallas import tpu as pltpu
from jax.experimental.pallas import tpu_sc as plsc

N = 10000
D = 256
DH = 128
E = 160000
E_PAD = 163840
ACC_ROWS = 10112
ROWS_PER_TILE = 632
NC, NS = 2, 16
CHUNK = 128
N_HALF = 2
HALF_CHUNKS = E_PAD // NS // N_HALF // CHUNK
DEG_CHUNKS = E_PAD // (NC * NS) // CHUNK
NBLK = 400
NSTEPS = N // NBLK

_mesh = plsc.VectorSubcoreMesh(core_axis_name="c", subcore_axis_name="s")

_acc_sd = jax.ShapeDtypeStruct((ACC_ROWS, DH), jnp.float32)


@functools.partial(
    pl.kernel, mesh=_mesh,
    out_type=(_acc_sd, _acc_sd),
    scratch_types=[
        pltpu.VMEM((DEG_CHUNKS, CHUNK), jnp.int32),
        pltpu.VMEM((CHUNK, DH), jnp.float32),
        pltpu.VMEM_SHARED((ACC_ROWS, DH), jnp.float32),
        pltpu.SemaphoreType.DMA,
    ],
)
def _deg_kernel(dst_hbm, ones_hbm, zeros_hbm, deg0_hbm, deg1_hbm,
                idx_v, ones_v, acc, dsem):
    c = lax.axis_index("c")
    s = lax.axis_index("s")
    wid = c * NS + s
    sl = pl.ds(s * ROWS_PER_TILE, ROWS_PER_TILE)
    pltpu.sync_copy(dst_hbm.at[wid], idx_v)
    pltpu.sync_copy(ones_hbm, ones_v)
    pltpu.sync_copy(zeros_hbm, acc.at[sl])
    plsc.subcore_barrier()

    @pl.loop(0, DEG_CHUNKS)
    def _(j):
        pltpu.async_copy(ones_v, acc.at[idx_v.at[j]], dsem, add=True)

    @pl.loop(0, DEG_CHUNKS)
    def _(j):
        pltpu.make_async_copy(ones_v, acc.at[idx_v.at[j]], dsem).wait()

    plsc.subcore_barrier()

    def out_copy(dref):
        pltpu.sync_copy(acc.at[sl], dref.at[sl])

    pl.when(c == 0)(lambda: out_copy(deg0_hbm))
    pl.when(c == 1)(lambda: out_copy(deg1_hbm))


@functools.partial(
    pl.kernel, mesh=_mesh,
    out_type=(_acc_sd, _acc_sd),
    scratch_types=[
        pltpu.VMEM((HALF_CHUNKS, CHUNK), jnp.int32),
        pltpu.VMEM((HALF_CHUNKS, CHUNK), jnp.int32),
        pltpu.VMEM((CHUNK, DH), jnp.float32),
        pltpu.VMEM((CHUNK, DH), jnp.float32),
        pltpu.VMEM_SHARED((ACC_ROWS, DH), jnp.float32),
        pltpu.SemaphoreType.DMA,
        pltpu.SemaphoreType.DMA,
        pltpu.SemaphoreType.DMA,
        pltpu.SemaphoreType.DMA,
    ],
)
def _agg_kernel(g0_hbm, g1_hbm, src_hbm, dst_hbm, zeros_hbm,
                out0_hbm, out1_hbm, idx_s, idx_d, buf0, buf1, acc,
                sem0, sem1, ssem0, ssem1):
    c = lax.axis_index("c")
    s = lax.axis_index("s")
    sl = pl.ds(s * ROWS_PER_TILE, ROWS_PER_TILE)
    pltpu.sync_copy(zeros_hbm, acc.at[sl])
    plsc.subcore_barrier()

    def run(g_hbm):
        @pl.loop(0, N_HALF)
        def _(h):
            pltpu.sync_copy(src_hbm.at[s * N_HALF + h], idx_s)
            pltpu.sync_copy(dst_hbm.at[s * N_HALF + h], idx_d)
            pltpu.async_copy(g_hbm.at[idx_s.at[0]], buf0, sem0)

            @pl.loop(0, HALF_CHUNKS, step=2)
            def _(j):
                pltpu.async_copy(g_hbm.at[idx_s.at[j + 1]], buf1, sem1)
                pltpu.make_async_copy(g_hbm.at[idx_s.at[j]], buf0, sem0).wait()
                pltpu.sync_copy(buf0, acc.at[idx_d.at[j]], add=True)

                @pl.when(j + 2 < HALF_CHUNKS)
                def _():
                    pltpu.async_copy(g_hbm.at[idx_s.at[j + 2]], buf0, sem0)

                pltpu.make_async_copy(g_hbm.at[idx_s.at[j + 1]], buf1,
                                      sem1).wait()
                pltpu.sync_copy(buf1, acc.at[idx_d.at[j + 1]], add=True)

    pl.when(c == 0)(lambda: run(g0_hbm))
    pl.when(c == 1)(lambda: run(g1_hbm))
    plsc.subcore_barrier()

    def out_copy(oref):
        pltpu.sync_copy(acc.at[sl], oref.at[sl])

    pl.when(c == 0)(lambda: out_copy(out0_hbm))
    pl.when(c == 1)(lambda: out_copy(out1_hbm))


def _dis_body(d0_ref, d1_ref, o_ref):
    deg = d0_ref[:, 0:1] + d1_ref[:, 0:1] + 1.0
    o_ref[...] = lax.rsqrt(deg)


def _mm1_body(x_ref, w_ref, dis_ref, g0_ref, g1_ref):
    h = jnp.dot(x_ref[...], w_ref[...], preferred_element_type=jnp.float32)
    g = h * dis_ref[...]
    g0_ref[...] = g[:, :DH]
    g1_ref[...] = g[:, DH:]


def _mid_body(a0_ref, a1_ref, g0_ref, g1_ref, dis_ref, b1_ref, w2_ref,
              o0_ref, o1_ref):
    agg = jnp.concatenate([a0_ref[...], a1_ref[...]], axis=1)
    g = jnp.concatenate([g0_ref[...], g1_ref[...]], axis=1)
    z = jnp.maximum(dis_ref[...] * (agg + g) + b1_ref[...], 0.0)
    h2 = jnp.dot(z, w2_ref[...], preferred_element_type=jnp.float32)
    g2 = h2 * dis_ref[...]
    o0_ref[...] = g2[:, :DH]
    o1_ref[...] = g2[:, DH:]


def _fin_body(a0_ref, a1_ref, g0_ref, g1_ref, dis_ref, b2_ref, o_ref):
    i = pl.program_id(0)
    agg = jnp.concatenate([a0_ref[...], a1_ref[...]], axis=1)
    g = jnp.concatenate([g0_ref[...], g1_ref[...]], axis=1)
    y = dis_ref[...] * (agg + g)
    part = jnp.sum(y, axis=0, keepdims=True)

    @pl.when(i == 0)
    def _():
        o_ref[...] = part

    @pl.when(i > 0)
    def _():
        o_ref[...] += part

    @pl.when(i == NSTEPS - 1)
    def _():
        o_ref[...] = o_ref[...] * (1.0 / N) + b2_ref[...]


_dis_call = pl.pallas_call(
    _dis_body,
    grid=(8,),
    in_specs=[
        pl.BlockSpec((ACC_ROWS // 8, DH), lambda i: (i, 0)),
        pl.BlockSpec((ACC_ROWS // 8, DH), lambda i: (i, 0)),
    ],
    out_specs=pl.BlockSpec((ACC_ROWS // 8, 1), lambda i: (i, 0)),
    out_shape=jax.ShapeDtypeStruct((ACC_ROWS, 1), jnp.float32),
)

_mm1_call = pl.pallas_call(
    _mm1_body,
    grid=(NSTEPS,),
    in_specs=[
        pl.BlockSpec((NBLK, D), lambda i: (i, 0)),
        pl.BlockSpec((D, D), lambda i: (0, 0)),
        pl.BlockSpec((NBLK, 1), lambda i: (i, 0)),
    ],
    out_specs=(
        pl.BlockSpec((NBLK, DH), lambda i: (i, 0)),
        pl.BlockSpec((NBLK, DH), lambda i: (i, 0)),
    ),
    out_shape=(
        jax.ShapeDtypeStruct((N, DH), jnp.float32),
        jax.ShapeDtypeStruct((N, DH), jnp.float32),
    ),
)

_mid_call = pl.pallas_call(
    _mid_body,
    grid=(NSTEPS,),
    in_specs=[
        pl.BlockSpec((NBLK, DH), lambda i: (i, 0)),
        pl.BlockSpec((NBLK, DH), lambda i: (i, 0)),
        pl.BlockSpec((NBLK, DH), lambda i: (i, 0)),
        pl.BlockSpec((NBLK, DH), lambda i: (i, 0)),
        pl.BlockSpec((NBLK, 1), lambda i: (i, 0)),
        pl.BlockSpec((1, D), lambda i: (0, 0)),
        pl.BlockSpec((D, D), lambda i: (0, 0)),
    ],
    out_specs=(
        pl.BlockSpec((NBLK, DH), lambda i: (i, 0)),
        pl.BlockSpec((NBLK, DH), lambda i: (i, 0)),
    ),
    out_shape=(
        jax.ShapeDtypeStruct((N, DH), jnp.float32),
        jax.ShapeDtypeStruct((N, DH), jnp.float32),
    ),
)

_fin_call = pl.pallas_call(
    _fin_body,
    grid=(NSTEPS,),
    in_specs=[
        pl.BlockSpec((NBLK, DH), lambda i: (i, 0)),
        pl.BlockSpec((NBLK, DH), lambda i: (i, 0)),
        pl.BlockSpec((NBLK, DH), lambda i: (i, 0)),
        pl.BlockSpec((NBLK, DH), lambda i: (i, 0)),
        pl.BlockSpec((NBLK, 1), lambda i: (i, 0)),
        pl.BlockSpec((1, D), lambda i: (0, 0)),
    ],
    out_specs=pl.BlockSpec((1, D), lambda i: (0, 0)),
    out_shape=jax.ShapeDtypeStruct((1, D), jnp.float32),
)


@jax.jit
def kernel(x, edge_index, W1, b1, W2, b2):
    src = edge_index[0].astype(jnp.int32)
    dst = edge_index[1].astype(jnp.int32)
    pad = E_PAD - E
    src_p = jnp.concatenate([src, jnp.zeros((pad,), jnp.int32)])
    dst_p = jnp.concatenate([dst, jnp.full((pad,), N, jnp.int32)])
    src_agg = src_p.reshape(NS * N_HALF, HALF_CHUNKS, CHUNK)
    dst_agg = dst_p.reshape(NS * N_HALF, HALF_CHUNKS, CHUNK)
    dst_deg = dst_p.reshape(NC * NS, DEG_CHUNKS, CHUNK)
    zeros = jnp.zeros((ROWS_PER_TILE, DH), jnp.float32)
    ones = jnp.ones((CHUNK, DH), jnp.float32)

    deg0, deg1 = _deg_kernel(dst_deg, ones, zeros)
    dis = _dis_call(deg0, deg1)

    g0, g1 = _mm1_call(x, W1, dis)
    a0, a1 = _agg_kernel(g0, g1, src_agg, dst_agg, zeros)
    h0, h1 = _mid_call(a0, a1, g0, g1, dis, b1.reshape(1, D), W2)
    c0, c1 = _agg_kernel(h0, h1, src_agg, dst_agg, zeros)
    out = _fin_call(c0, c1, h0, h1, dis, b2.reshape(1, D))
    return out
